# Initial kernel scaffold; baseline (speedup 1.0000x reference)
import jax
import jax.numpy as jnp
from jax.experimental import pallas as pl


def kernel(input_ids, embed, layers, final_norm, lm_head):
    raise NotImplementedError("write your pallas kernel here")


# speedup vs baseline: 1.0011x; 1.0011x over previous
import functools
import math

import jax
import jax.numpy as jnp
import numpy as np
from jax.experimental import pallas as pl

_B, _S, _D, _H, _V, _FF, _NL = 2, 2048, 1024, 16, 16384, 4096, 2
_DH = _D // _H
_CF = 0.5
_AUX_COEFF = 0.01
_KCAP = max(1, math.ceil(_CF * _S))


def _freqs(seq_len, dh, theta=10000.0):
    inv = 1.0 / (theta ** (np.arange(0, dh, 2, dtype=np.float64) / dh))
    t = np.arange(seq_len, dtype=np.float64)
    f = np.outer(t, inv)
    return (jnp.asarray(np.cos(f), dtype=jnp.float32),
            jnp.asarray(np.sin(f), dtype=jnp.float32))


def _rmsnorm(x, w, eps=1e-6):
    return x * jax.lax.rsqrt(jnp.mean(x * x, axis=-1, keepdims=True) + eps) * w


def _rope(x, cos, sin):
    x1 = x[..., 0::2]
    x2 = x[..., 1::2]
    c = cos[None, :, None, :]
    s = sin[None, :, None, :]
    r1 = x1 * c - x2 * s
    r2 = x1 * s + x2 * c
    return jnp.stack([r1, r2], axis=-1).reshape(x.shape)


def _dense_block(x, p, cos, sin):
    Bk, T, _ = x.shape
    h = _rmsnorm(x, p['norm1'])
    q = (h @ p['Wq']).reshape(Bk, T, _H, _DH)
    k = (h @ p['Wk']).reshape(Bk, T, _H, _DH)
    v = (h @ p['Wv']).reshape(Bk, T, _H, _DH)
    q = _rope(q, cos[:T], sin[:T])
    k = _rope(k, cos[:T], sin[:T])
    q = q.transpose(0, 2, 1, 3)
    k = k.transpose(0, 2, 1, 3)
    v = v.transpose(0, 2, 1, 3)
    att = jnp.einsum('bhqd,bhkd->bhqk', q, k) / math.sqrt(_DH)
    att = jax.nn.softmax(att, axis=-1)
    o = jnp.einsum('bhqk,bhkd->bhqd', att, v)
    o = o.transpose(0, 2, 1, 3).reshape(Bk, T, _D)
    x = x + o @ p['Wo']
    h2 = _rmsnorm(x, p['norm2'])
    x = x + jax.nn.silu(h2 @ p['W1']) @ p['W2']
    return x


def _loss_passthrough_kernel(l_ref, o_ref):
    o_ref[...] = l_ref[...]


def kernel(input_ids, embed, layers, final_norm, lm_head):
    cos, sin = _freqs(_S, _DH)
    x = embed[input_ids]
    total_aux = jnp.zeros((), x.dtype)
    for p in layers:
        scores = jnp.einsum('btd,d->bt', x, p['gate'])
        _vals, idx = jax.lax.top_k(scores, _KCAP)
        sel = jnp.take_along_axis(x, idx[:, :, None], axis=1)
        out = _dense_block(sel, p, cos, sin)
        x = x.at[jnp.arange(_B)[:, None], idx].set(out)
        probs = jax.nn.sigmoid(scores)
        total_aux = total_aux + jnp.mean((jnp.mean(probs, axis=1) - _CF) ** 2)
    x = _rmsnorm(x, final_norm)
    logits = x @ lm_head
    loss_in = (_AUX_COEFF * total_aux).reshape(1, 1)
    loss = pl.pallas_call(
        _loss_passthrough_kernel,
        out_shape=jax.ShapeDtypeStruct((1, 1), jnp.float32),
    )(loss_in).reshape(())
    return loss, logits


# speedup vs baseline: 1.0407x; 1.0396x over previous
import functools
import math

import jax
import jax.numpy as jnp
import numpy as np
from jax.experimental import pallas as pl
from jax.experimental.pallas import tpu as pltpu

_B, _S, _D, _H, _V, _FF, _NL = 2, 2048, 1024, 16, 16384, 4096, 2
_DH = _D // _H
_CF = 0.5
_AUX_COEFF = 0.01
_KCAP = max(1, math.ceil(_CF * _S))
_FFC = 1024
_NFC = _FF // _FFC


def _freqs(seq_len, dh, theta=10000.0):
    inv = 1.0 / (theta ** (np.arange(0, dh, 2, dtype=np.float64) / dh))
    t = np.arange(seq_len, dtype=np.float64)
    f = np.outer(t, inv)
    return np.cos(f), np.sin(f)


@functools.lru_cache(maxsize=None)
def _rope_tables():
    cos, sin = _freqs(_KCAP, _DH)
    lane = np.arange(_D)
    fi = (lane % _DH) // 2
    cos_big = np.asarray(cos, np.float32)[:, fi]
    sin_big = np.asarray(sin, np.float32)[:, fi]
    sign = np.where(lane % 2 == 0, -1.0, 1.0).astype(np.float32)
    sin_signed = sin_big * sign[None, :]
    perm = np.zeros((_D, _D), np.float32)
    perm[lane ^ 1, lane] = 1.0
    return cos_big, sin_signed, perm


def _rms_rows(x, w, eps=1e-6):
    return x * jax.lax.rsqrt(jnp.mean(x * x, axis=-1, keepdims=True) + eps) * w


def _qkv_kernel(sel_ref, wq_ref, wk_ref, wv_ref, n1_ref, cos_ref, sin_ref,
                perm_ref, q_ref, k_ref, v_ref):
    h = _rms_rows(sel_ref[0], n1_ref[...]).astype(jnp.bfloat16)
    d = lambda a, b: jnp.dot(a, b.astype(jnp.bfloat16),
                             preferred_element_type=jnp.float32)
    q = d(h, wq_ref[...])
    k = d(h, wk_ref[...])
    v = d(h, wv_ref[...])
    cos = cos_ref[...]
    sin = sin_ref[...]
    even = (jax.lax.broadcasted_iota(jnp.int32, (_QT, _D), 1) % 2) == 0
    swap = lambda t: jnp.where(even, pltpu.roll(t, _D - 1, 1), pltpu.roll(t, 1, 1))
    del perm_ref
    q_ref[0] = (q * cos + swap(q) * sin).astype(jnp.bfloat16)
    k_ref[0] = (k * cos + swap(k) * sin).astype(jnp.bfloat16)
    v_ref[0] = v


_QT = 512


def _qkv_call(sel, p):
    cos_big, sin_signed, perm = _rope_tables()
    full = lambda b, t: (0, 0)
    return pl.pallas_call(
        _qkv_kernel,
        grid=(_B, _KCAP // _QT),
        in_specs=[
            pl.BlockSpec((1, _QT, _D), lambda b, t: (b, t, 0)),
            pl.BlockSpec((_D, _D), full),
            pl.BlockSpec((_D, _D), full),
            pl.BlockSpec((_D, _D), full),
            pl.BlockSpec((1, _D), full),
            pl.BlockSpec((_QT, _D), lambda b, t: (t, 0)),
            pl.BlockSpec((_QT, _D), lambda b, t: (t, 0)),
            pl.BlockSpec((_D, _D), full),
        ],
        out_specs=[
            pl.BlockSpec((1, _QT, _D), lambda b, t: (b, t, 0)),
            pl.BlockSpec((1, _QT, _D), lambda b, t: (b, t, 0)),
            pl.BlockSpec((1, _QT, _D), lambda b, t: (b, t, 0)),
        ],
        out_shape=[
            jax.ShapeDtypeStruct((_B, _KCAP, _D), jnp.bfloat16),
            jax.ShapeDtypeStruct((_B, _KCAP, _D), jnp.bfloat16),
            jax.ShapeDtypeStruct((_B, _KCAP, _D), jnp.float32),
        ],
    )(sel, p['Wq'], p['Wk'], p['Wv'], p['norm1'].reshape(1, _D),
      jnp.asarray(cos_big), jnp.asarray(sin_signed), jnp.asarray(perm))


def _attn_kernel(q_ref, k_ref, v_ref, o_ref):
    q = q_ref[0]
    k = k_ref[0]
    s = jax.lax.dot_general(q, k, (((1,), (1,)), ((), ())),
                            preferred_element_type=jnp.float32)
    s = s * (1.0 / math.sqrt(_DH))
    m = jnp.max(s, axis=-1, keepdims=True)
    e = jnp.exp(s - m)
    o = jnp.dot(e.astype(jnp.bfloat16), v_ref[0].astype(jnp.bfloat16),
                preferred_element_type=jnp.float32)
    o_ref[0] = o / jnp.sum(e, axis=-1, keepdims=True)


def _attn_call(qh, kh, vh):
    return pl.pallas_call(
        _attn_kernel,
        grid=(_B * _H,),
        in_specs=[
            pl.BlockSpec((1, _KCAP, _DH), lambda i: (i, 0, 0)),
            pl.BlockSpec((1, _KCAP, _DH), lambda i: (i, 0, 0)),
            pl.BlockSpec((1, _KCAP, _DH), lambda i: (i, 0, 0)),
        ],
        out_specs=pl.BlockSpec((1, _KCAP, _DH), lambda i: (i, 0, 0)),
        out_shape=jax.ShapeDtypeStruct((_B * _H, _KCAP, _DH), jnp.float32),
    )(qh, kh, vh)


_MT = 512


def _mlp_kernel(sel_ref, o_ref, wo_ref, n2_ref, w1_ref, w2_ref, y_ref,
                x2_sc, h2_sc, acc_sc):
    c = pl.program_id(2)

    @pl.when(c == 0)
    def _init():
        x2 = sel_ref[0] + jnp.dot(o_ref[0], wo_ref[...].astype(jnp.bfloat16),
                                  preferred_element_type=jnp.float32)
        x2_sc[...] = x2
        h2_sc[...] = _rms_rows(x2, n2_ref[...])
        acc_sc[...] = x2

    u = jnp.dot(h2_sc[...].astype(jnp.bfloat16), w1_ref[...].astype(jnp.bfloat16),
                preferred_element_type=jnp.float32)
    u = (u * jax.nn.sigmoid(u)).astype(jnp.bfloat16)
    acc_sc[...] += jnp.dot(u, w2_ref[...].astype(jnp.bfloat16),
                           preferred_element_type=jnp.float32)

    @pl.when(c == _NFC - 1)
    def _fin():
        y_ref[0] = acc_sc[...]


def _mlp_call(sel, o_bf16, p):
    return pl.pallas_call(
        _mlp_kernel,
        grid=(_B, _KCAP // _MT, _NFC),
        in_specs=[
            pl.BlockSpec((1, _MT, _D), lambda b, t, c: (b, t, 0)),
            pl.BlockSpec((1, _MT, _D), lambda b, t, c: (b, t, 0)),
            pl.BlockSpec((_D, _D), lambda b, t, c: (0, 0)),
            pl.BlockSpec((1, _D), lambda b, t, c: (0, 0)),
            pl.BlockSpec((_D, _FFC), lambda b, t, c: (0, c)),
            pl.BlockSpec((_FFC, _D), lambda b, t, c: (c, 0)),
        ],
        out_specs=pl.BlockSpec((1, _MT, _D), lambda b, t, c: (b, t, 0)),
        out_shape=jax.ShapeDtypeStruct((_B, _KCAP, _D), jnp.float32),
        scratch_shapes=[
            pltpu.VMEM((_MT, _D), jnp.float32),
            pltpu.VMEM((_MT, _D), jnp.float32),
            pltpu.VMEM((_MT, _D), jnp.float32),
        ],
    )(sel, o_bf16, p['Wo'], p['norm2'].reshape(1, _D), p['W1'], p['W2'])


_LM_TM = 1024
_LM_TN = 1024


def _lm_kernel(x_ref, fn_ref, w_ref, o_ref):
    h = _rms_rows(x_ref[...], fn_ref[...])
    o_ref[...] = jnp.dot(h.astype(jnp.bfloat16), w_ref[...].astype(jnp.bfloat16),
                         preferred_element_type=jnp.float32)


def _lm_call(xf, final_norm, lm_head):
    return pl.pallas_call(
        _lm_kernel,
        grid=(_B * _S // _LM_TM, _V // _LM_TN),
        in_specs=[
            pl.BlockSpec((_LM_TM, _D), lambda m, n: (m, 0)),
            pl.BlockSpec((1, _D), lambda m, n: (0, 0)),
            pl.BlockSpec((_D, _LM_TN), lambda m, n: (0, n)),
        ],
        out_specs=pl.BlockSpec((_LM_TM, _LM_TN), lambda m, n: (m, n)),
        out_shape=jax.ShapeDtypeStruct((_B * _S, _V), jnp.float32),
    )(xf, final_norm.reshape(1, _D), lm_head)


def _loss_passthrough_kernel(l_ref, o_ref):
    o_ref[...] = l_ref[...]


def _rot_ref(x, cos, sin):
    x1 = x[..., 0::2]
    x2 = x[..., 1::2]
    c = cos[None, :, None, :]
    s = sin[None, :, None, :]
    r1 = x1 * c - x2 * s
    r2 = x1 * s + x2 * c
    return jnp.stack([r1, r2], axis=-1).reshape(x.shape)


def _qkv_jnp(sel, p):
    cosn, sinn = _freqs(_KCAP, _DH)
    cos = jnp.asarray(cosn, jnp.float32)
    sin = jnp.asarray(sinn, jnp.float32)
    h = sel * jax.lax.rsqrt(jnp.mean(sel * sel, axis=-1, keepdims=True)
                            + 1e-6) * p['norm1']
    q = (h @ p['Wq']).reshape(_B, _KCAP, _H, _DH)
    k = (h @ p['Wk']).reshape(_B, _KCAP, _H, _DH)
    v = (h @ p['Wv']).reshape(_B, _KCAP, _H, _DH)
    q = _rot_ref(q, cos, sin).astype(jnp.bfloat16).reshape(_B, _KCAP, _D)
    k = _rot_ref(k, cos, sin).astype(jnp.bfloat16).reshape(_B, _KCAP, _D)
    return q, k, v.reshape(_B, _KCAP, _D)


def _dense_block(sel, p):
    q, k, v = _qkv_jnp(sel, p)
    to_h = lambda t: (t.reshape(_B, _KCAP, _H, _DH).transpose(0, 2, 1, 3)
                      .reshape(_B * _H, _KCAP, _DH))
    o = _attn_call(to_h(q), to_h(k), to_h(v))
    o = (o.reshape(_B, _H, _KCAP, _DH).transpose(0, 2, 1, 3)
         .reshape(_B, _KCAP, _D))
    return _mlp_call(sel, o.astype(jnp.bfloat16), p)


def kernel(input_ids, embed, layers, final_norm, lm_head):
    x = embed[input_ids]
    total_aux = jnp.zeros((), x.dtype)
    for p in layers:
        scores = jnp.einsum('btd,d->bt', x, p['gate'])
        _vals, idx = jax.lax.top_k(scores, _KCAP)
        sel = jnp.take_along_axis(x, idx[:, :, None], axis=1)
        out = _dense_block(sel, p)
        x = x.at[jnp.arange(_B)[:, None], idx].set(out)
        probs = jax.nn.sigmoid(scores)
        total_aux = total_aux + jnp.mean((jnp.mean(probs, axis=1) - _CF) ** 2)
    logits = _lm_call(x.reshape(_B * _S, _D), final_norm,
                      lm_head).reshape(_B, _S, _V)
    loss_in = (_AUX_COEFF * total_aux).reshape(1, 1)
    loss = pl.pallas_call(
        _loss_passthrough_kernel,
        out_shape=jax.ShapeDtypeStruct((1, 1), jnp.float32),
    )(loss_in).reshape(())
    return loss, logits
